# Initial kernel scaffold; baseline (speedup 1.0000x reference)
import functools

import jax
import jax.numpy as jnp
from jax import lax
from jax.experimental import pallas as pl
from jax.experimental.pallas import tpu as pltpu
from jax.experimental.pallas import tpu_sc as plsc

VOCAB = 100000
EMBED = 50
BATCH = 4096
SEQ = 50

NUM_CORES = 2
NUM_SUBCORES = 16
NUM_WORKERS = NUM_CORES * NUM_SUBCORES

B_TOTAL = BATCH * SEQ
PER_W = B_TOTAL // NUM_WORKERS
CHUNK = 128
GROUP = 5
GROUP_ROWS = GROUP * CHUNK
NGROUPS = PER_W // GROUP_ROWS


def _gather_body(idx_hbm, table_hbm, out_hbm, idx_v, buf, sem_g):
    cid = lax.axis_index("c")
    sid = lax.axis_index("s")
    wid = sid * NUM_CORES + cid
    base = wid * PER_W

    pltpu.sync_copy(idx_hbm.at[pl.ds(base, PER_W)], idx_v)

    def group(g):
        off = g * GROUP_ROWS
        copies = []
        for j in range(GROUP):
            copies.append(
                pltpu.async_copy(
                    table_hbm.at[idx_v.at[pl.ds(off + j * CHUNK, CHUNK)]],
                    buf.at[pl.ds(j * CHUNK, CHUNK), :],
                    sem_g,
                )
            )
        for c in copies:
            c.wait()
        pltpu.sync_copy(buf, out_hbm.at[pl.ds(base + off, GROUP_ROWS), :])

    pl.loop(0, NGROUPS)(group)


@jax.jit
def _run(idx_flat, table):
    kern = pl.kernel(
        _gather_body,
        out_type=jax.ShapeDtypeStruct((B_TOTAL, EMBED), jnp.float32),
        mesh=plsc.VectorSubcoreMesh(core_axis_name="c", subcore_axis_name="s"),
        scratch_types=[
            pltpu.VMEM((PER_W,), jnp.int32),
            pltpu.VMEM((GROUP_ROWS, EMBED), jnp.float32),
            pltpu.SemaphoreType.DMA,
        ],
    )
    return kern(idx_flat, table)


def kernel(indices, table):
    idx_flat = indices.reshape(B_TOTAL).astype(jnp.int32)
    out = _run(idx_flat, table)
    return out.reshape(BATCH, SEQ, EMBED)


# speedup vs baseline: 2.5515x; 2.5515x over previous
import functools

import jax
import jax.numpy as jnp
from jax import lax
from jax.experimental import pallas as pl
from jax.experimental.pallas import tpu as pltpu
from jax.experimental.pallas import tpu_sc as plsc

VOCAB = 100000
EMBED = 50
EMBED_P = 56
BATCH = 4096
SEQ = 50

NUM_CORES = 2
NUM_SUBCORES = 16
NUM_WORKERS = NUM_CORES * NUM_SUBCORES

B_TOTAL = BATCH * SEQ
PER_W = B_TOTAL // NUM_WORKERS
CHUNK = 128
GROUP = 5
GROUP_ROWS = GROUP * CHUNK
NGROUPS = PER_W // GROUP_ROWS


def _gather_body(idx_hbm, table_hbm, out_hbm, idx_v, buf, sem_g):
    cid = lax.axis_index("c")
    sid = lax.axis_index("s")
    wid = sid * NUM_CORES + cid
    base = wid * PER_W

    pltpu.sync_copy(idx_hbm.at[pl.ds(base, PER_W)], idx_v)

    def group(g):
        off = g * GROUP_ROWS
        copies = []
        for j in range(GROUP):
            copies.append(
                pltpu.async_copy(
                    table_hbm.at[idx_v.at[pl.ds(off + j * CHUNK, CHUNK)]],
                    buf.at[pl.ds(j * CHUNK, CHUNK), :],
                    sem_g,
                )
            )
        for c in copies:
            c.wait()
        pltpu.sync_copy(buf, out_hbm.at[pl.ds(base + off, GROUP_ROWS), :])

    pl.loop(0, NGROUPS)(group)


@jax.jit
def _run(idx_flat, table):
    kern = pl.kernel(
        _gather_body,
        out_type=jax.ShapeDtypeStruct((B_TOTAL, EMBED_P), jnp.float32),
        mesh=plsc.VectorSubcoreMesh(core_axis_name="c", subcore_axis_name="s"),
        scratch_types=[
            pltpu.VMEM((PER_W,), jnp.int32),
            pltpu.VMEM((GROUP_ROWS, EMBED_P), jnp.float32),
            pltpu.SemaphoreType.DMA,
        ],
        compiler_params=pltpu.CompilerParams(use_tc_tiling_on_sc=False),
    )
    return kern(idx_flat, table)


def kernel(indices, table):
    idx_flat = indices.reshape(B_TOTAL).astype(jnp.int32)
    table_p = jnp.pad(table, ((0, 0), (0, EMBED_P - EMBED)))
    out = _run(idx_flat, table_p)
    return out[:, :EMBED].reshape(BATCH, SEQ, EMBED)


# speedup vs baseline: 3.2549x; 1.2757x over previous
import jax
import jax.numpy as jnp
from jax import lax
from jax.experimental import pallas as pl
from jax.experimental.pallas import tpu as pltpu
from jax.experimental.pallas import tpu_sc as plsc

VOCAB = 100000
EMBED = 50
EMBED_P = 56
BATCH = 4096
SEQ = 50

NUM_CORES = 2
NUM_SUBCORES = 16
NUM_WORKERS = NUM_CORES * NUM_SUBCORES

BPT = BATCH // NUM_WORKERS
G = 4
NGROUPS = BPT // G


def _gather_body(idx_hbm, table_hbm, out_hbm, idx_v, buf, sem_g):
    cid = lax.axis_index("c")
    sid = lax.axis_index("s")
    wid = sid * NUM_CORES + cid
    b0 = wid * BPT

    pltpu.sync_copy(idx_hbm.at[pl.ds(b0, BPT), :], idx_v)

    def group(g):
        copies = []
        for k in range(G):
            copies.append(
                pltpu.async_copy(
                    table_hbm.at[idx_v.at[g * G + k, :]],
                    buf.at[k],
                    sem_g,
                )
            )
        for c in copies:
            c.wait()
        pltpu.sync_copy(buf, out_hbm.at[pl.ds(b0 + g * G, G), :, :])

    pl.loop(0, NGROUPS)(group)


@jax.jit
def _run(idx, table_p):
    kern = pl.kernel(
        _gather_body,
        out_type=jax.ShapeDtypeStruct((BATCH, SEQ, EMBED_P), jnp.float32),
        mesh=plsc.VectorSubcoreMesh(core_axis_name="c", subcore_axis_name="s"),
        scratch_types=[
            pltpu.VMEM((BPT, SEQ), jnp.int32),
            pltpu.VMEM((G, SEQ, EMBED_P), jnp.float32),
            pltpu.SemaphoreType.DMA,
        ],
        compiler_params=pltpu.CompilerParams(use_tc_tiling_on_sc=False),
    )
    return kern(idx, table_p)


def kernel(indices, table):
    idx = indices.astype(jnp.int32)
    table_p = jnp.pad(table, ((0, 0), (0, EMBED_P - EMBED)))
    out = _run(idx, table_p)
    return out[:, :, :EMBED]
